# Initial kernel scaffold; baseline (speedup 1.0000x reference)
import jax
import jax.numpy as jnp
from jax.experimental import pallas as pl


def kernel(x, edge_index, batch, W1, b1, W2, b2, Wfc, bfc):
    raise NotImplementedError("write your pallas kernel here")


# speedup vs baseline: 6.5073x; 6.5073x over previous
import functools

import jax
import jax.numpy as jnp
from jax import lax
from jax.experimental import pallas as pl
from jax.experimental.pallas import tpu as pltpu
from jax.experimental.pallas import tpu_sc as plsc

N = 10000
E = 160000
D = 256
NCLS = 10
NG = 64
NP = 10240
EP = 163840
HD = 128
BR = 512
GRID = NP // BR

_mesh = plsc.VectorSubcoreMesh(core_axis_name="c", subcore_axis_name="s")


@functools.partial(
    pl.kernel,
    out_type=jax.ShapeDtypeStruct((2, NP, 16), jnp.float32),
    mesh=_mesh,
    scratch_types=[
        pltpu.VMEM_SHARED((NP, 16), jnp.float32),
        pltpu.VMEM((128, 16), jnp.float32),
        pltpu.VMEM((640, 16), jnp.float32),
        pltpu.VMEM((1, 128), jnp.int32),
    ],
)
def _deg_kernel(dst_hbm, dp_hbm, acc, ones_v, wbuf, idx_v):
    c = lax.axis_index("c")
    s = lax.axis_index("s")

    @pl.loop(0, 640)
    def _(i):
        wbuf.at[pl.ds(i, 1), :][...] = jnp.zeros((1, 16), jnp.float32)

    @pl.loop(0, 128)
    def _(i):
        ones_v.at[pl.ds(i, 1), :][...] = jnp.ones((1, 16), jnp.float32)

    pltpu.sync_copy(wbuf, acc.at[pl.ds(s * 640, 640)])
    plsc.subcore_barrier()

    base = (c * 16 + s) * 5120
    @pl.loop(0, 40)
    def _(k):
        pltpu.sync_copy(dst_hbm.at[pl.ds(base + k * 128, 128)], idx_v.at[0])
        pltpu.sync_copy(ones_v, acc.at[idx_v.at[0]], add=True)

    plsc.subcore_barrier()
    pltpu.sync_copy(acc.at[pl.ds(s * 640, 640)], wbuf)
    pltpu.sync_copy(wbuf, dp_hbm.at[c].at[pl.ds(s * 640, 640)])


@functools.partial(
    pl.kernel,
    out_type=[jax.ShapeDtypeStruct((NP, HD), jnp.float32),
              jax.ShapeDtypeStruct((NP, HD), jnp.float32)],
    mesh=_mesh,
    scratch_types=[
        pltpu.VMEM_SHARED((NP, HD), jnp.float32),
        pltpu.VMEM((64, HD), jnp.float32),
        pltpu.VMEM((128, HD), jnp.float32),
        pltpu.VMEM((1, 128), jnp.int32),
        pltpu.VMEM((1, 128), jnp.int32),
    ],
)
def _agg_kernel(ya_hbm, yb_hbm, src_hbm, dst_hbm, oa_hbm, ob_hbm,
                acc, zv, rows, isv, idv):
    c = lax.axis_index("c")
    s = lax.axis_index("s")

    @pl.loop(0, 64)
    def _(i):
        @pl.loop(0, HD, step=16)
        def _(j):
            zv.at[pl.ds(i, 1), pl.ds(j, 16)][...] = jnp.zeros((1, 16), jnp.float32)

    @pl.loop(0, 10)
    def _(t):
        pltpu.sync_copy(zv, acc.at[pl.ds(s * 640 + t * 64, 64)])
    plsc.subcore_barrier()

    ebase = s * 10240

    def run(tab, out):
        @pl.loop(0, 80)
        def _(k):
            off = ebase + k * 128
            pltpu.sync_copy(src_hbm.at[pl.ds(off, 128)], isv.at[0])
            pltpu.sync_copy(dst_hbm.at[pl.ds(off, 128)], idv.at[0])
            pltpu.sync_copy(tab.at[isv.at[0]], rows)
            pltpu.sync_copy(rows, acc.at[idv.at[0]], add=True)
        plsc.subcore_barrier()

        @pl.loop(0, 5)
        def _(t):
            r0 = s * 640 + t * 128
            pltpu.sync_copy(acc.at[pl.ds(r0, 128)], rows)
            pltpu.sync_copy(rows, out.at[pl.ds(r0, 128)])

    @pl.when(c == 0)
    def _():
        run(ya_hbm, oa_hbm)

    @pl.when(c == 1)
    def _():
        run(yb_hbm, ob_hbm)


def _lin1_body(x_ref, dp_ref, w_ref, ya_ref, yb_ref, dinv_ref):
    deg = dp_ref[0][:, 0:1] + dp_ref[1][:, 0:1] + 1.0
    dinv = lax.rsqrt(deg)
    xw = jnp.dot(x_ref[...], w_ref[...], preferred_element_type=jnp.float32)
    y = xw * dinv
    ya_ref[...] = y[:, :HD]
    yb_ref[...] = y[:, HD:]
    dinv_ref[...] = dinv


_lin1 = pl.pallas_call(
    _lin1_body,
    grid=(GRID,),
    in_specs=[
        pl.BlockSpec((BR, D), lambda i: (i, 0)),
        pl.BlockSpec((2, BR, 16), lambda i: (0, i, 0)),
        pl.BlockSpec((D, D), lambda i: (0, 0)),
    ],
    out_specs=[
        pl.BlockSpec((BR, HD), lambda i: (i, 0)),
        pl.BlockSpec((BR, HD), lambda i: (i, 0)),
        pl.BlockSpec((BR, 1), lambda i: (i, 0)),
    ],
    out_shape=[
        jax.ShapeDtypeStruct((NP, HD), jnp.float32),
        jax.ShapeDtypeStruct((NP, HD), jnp.float32),
        jax.ShapeDtypeStruct((NP, 1), jnp.float32),
    ],
)


def _lin2_body(aa_ref, ab_ref, ya_ref, yb_ref, dinv_ref, b1_ref, w2_ref,
               oa_ref, ob_ref):
    dinv = dinv_ref[...]
    g = jnp.concatenate([aa_ref[...] + ya_ref[...],
                         ab_ref[...] + yb_ref[...]], axis=1)
    h = jnp.maximum(g * dinv + b1_ref[...][None, :], 0.0)
    xw = jnp.dot(h, w2_ref[...], preferred_element_type=jnp.float32)
    y2 = xw * dinv
    oa_ref[...] = y2[:, :HD]
    ob_ref[...] = y2[:, HD:]


_lin2 = pl.pallas_call(
    _lin2_body,
    grid=(GRID,),
    in_specs=[
        pl.BlockSpec((BR, HD), lambda i: (i, 0)),
        pl.BlockSpec((BR, HD), lambda i: (i, 0)),
        pl.BlockSpec((BR, HD), lambda i: (i, 0)),
        pl.BlockSpec((BR, HD), lambda i: (i, 0)),
        pl.BlockSpec((BR, 1), lambda i: (i, 0)),
        pl.BlockSpec((D,), lambda i: (0,)),
        pl.BlockSpec((D, D), lambda i: (0, 0)),
    ],
    out_specs=[
        pl.BlockSpec((BR, HD), lambda i: (i, 0)),
        pl.BlockSpec((BR, HD), lambda i: (i, 0)),
    ],
    out_shape=[
        jax.ShapeDtypeStruct((NP, HD), jnp.float32),
        jax.ShapeDtypeStruct((NP, HD), jnp.float32),
    ],
)


def _pool_body(aa_ref, ab_ref, ya_ref, yb_ref, dinv_ref, b2_ref, batch_ref,
               out_ref):
    i = pl.program_id(0)
    dinv = dinv_ref[...]
    g = jnp.concatenate([aa_ref[...] + ya_ref[...],
                         ab_ref[...] + yb_ref[...]], axis=1)
    h = g * dinv + b2_ref[...][None, :]
    bt = batch_ref[...]
    m = (bt[None, :] == lax.broadcasted_iota(jnp.int32, (NG, BR), 0)
         ).astype(jnp.float32)

    @pl.when(i == 0)
    def _():
        out_ref[...] = jnp.zeros_like(out_ref)

    out_ref[...] += jnp.dot(m, h, preferred_element_type=jnp.float32)


_pool = pl.pallas_call(
    _pool_body,
    grid=(GRID,),
    in_specs=[
        pl.BlockSpec((BR, HD), lambda i: (i, 0)),
        pl.BlockSpec((BR, HD), lambda i: (i, 0)),
        pl.BlockSpec((BR, HD), lambda i: (i, 0)),
        pl.BlockSpec((BR, HD), lambda i: (i, 0)),
        pl.BlockSpec((BR, 1), lambda i: (i, 0)),
        pl.BlockSpec((D,), lambda i: (0,)),
        pl.BlockSpec((BR,), lambda i: (i,)),
    ],
    out_specs=pl.BlockSpec((NG, D), lambda i: (0, 0)),
    out_shape=jax.ShapeDtypeStruct((NG, D), jnp.float32),
)


def _head_body(p_ref, wfc_ref, bfc_ref, out_ref):
    z = jnp.dot(p_ref[...], wfc_ref[...], preferred_element_type=jnp.float32)
    z = z + bfc_ref[...][None, :]
    mx = jnp.max(z, axis=1, keepdims=True)
    lse = jnp.log(jnp.sum(jnp.exp(z - mx), axis=1, keepdims=True)) + mx
    out_ref[...] = z - lse


_head = pl.pallas_call(
    _head_body,
    grid=(1,),
    in_specs=[
        pl.BlockSpec((NG, D), lambda i: (0, 0)),
        pl.BlockSpec((D, NCLS), lambda i: (0, 0)),
        pl.BlockSpec((NCLS,), lambda i: (0,)),
    ],
    out_specs=pl.BlockSpec((NG, NCLS), lambda i: (0, 0)),
    out_shape=jax.ShapeDtypeStruct((NG, NCLS), jnp.float32),
)


def kernel(x, edge_index, batch, W1, b1, W2, b2, Wfc, bfc):
    src = edge_index[0]
    dst = edge_index[1]
    pad_e = EP - E
    src_p = jnp.concatenate([src, jnp.zeros((pad_e,), jnp.int32)])
    dst_p = jnp.concatenate(
        [dst, N + (jnp.arange(pad_e, dtype=jnp.int32) % (NP - N))])
    x_p = jnp.pad(x, ((0, NP - N), (0, 0)))
    batch_p = jnp.pad(batch, (0, NP - N), constant_values=NG)

    dp = _deg_kernel(dst_p)
    ya, yb, dinv = _lin1(x_p, dp, W1)
    a1a, a1b = _agg_kernel(ya, yb, src_p, dst_p)
    y2a, y2b = _lin2(a1a, a1b, ya, yb, dinv, b1, W2)
    a2a, a2b = _agg_kernel(y2a, y2b, src_p, dst_p)
    pooled = _pool(a2a, a2b, y2a, y2b, dinv, b2, batch_p)
    return _head(pooled, Wfc, bfc)


# speedup vs baseline: 8.0139x; 1.2315x over previous
import functools

import jax
import jax.numpy as jnp
from jax import lax
from jax.experimental import pallas as pl
from jax.experimental.pallas import tpu as pltpu
from jax.experimental.pallas import tpu_sc as plsc

N = 10000
E = 160000
D = 256
NCLS = 10
NG = 64
NP = 10240
EP = 163840
HD = 128
BR = 512
GRID = NP // BR

_mesh = plsc.VectorSubcoreMesh(core_axis_name="c", subcore_axis_name="s")


@functools.partial(
    pl.kernel,
    out_type=jax.ShapeDtypeStruct((2, NP, 16), jnp.float32),
    mesh=_mesh,
    scratch_types=[
        pltpu.VMEM_SHARED((NP, 16), jnp.float32),
        pltpu.VMEM((128, 16), jnp.float32),
        pltpu.VMEM((640, 16), jnp.float32),
        pltpu.VMEM((40, 128), jnp.int32),
        pltpu.SemaphoreType.DMA,
    ],
)
def _deg_kernel(dst_hbm, dp_hbm, acc, ones_v, wbuf, idx_v, sem):
    c = lax.axis_index("c")
    s = lax.axis_index("s")

    @pl.loop(0, 640)
    def _(i):
        wbuf.at[pl.ds(i, 1), :][...] = jnp.zeros((1, 16), jnp.float32)

    @pl.loop(0, 128)
    def _(i):
        ones_v.at[pl.ds(i, 1), :][...] = jnp.ones((1, 16), jnp.float32)

    pltpu.sync_copy(dst_hbm.at[c * 16 + s], idx_v)
    pltpu.sync_copy(wbuf, acc.at[pl.ds(s * 640, 640)])
    plsc.subcore_barrier()

    @pl.loop(0, 40, step=8)
    def _(k0):
        cps = [pltpu.async_copy(ones_v, acc.at[idx_v.at[k0 + j]], sem, add=True)
               for j in range(8)]
        for cp in cps:
            cp.wait()

    plsc.subcore_barrier()
    pltpu.sync_copy(acc.at[pl.ds(s * 640, 640)], wbuf)
    pltpu.sync_copy(wbuf, dp_hbm.at[c].at[pl.ds(s * 640, 640)])


NBUF = 2


@functools.partial(
    pl.kernel,
    out_type=[jax.ShapeDtypeStruct((NP, HD), jnp.float32),
              jax.ShapeDtypeStruct((NP, HD), jnp.float32)],
    mesh=_mesh,
    scratch_types=[
        pltpu.VMEM_SHARED((NP, HD), jnp.float32),
        pltpu.VMEM((16, HD), jnp.float32),
        pltpu.VMEM((NBUF, 128, HD), jnp.float32),
        pltpu.VMEM((40, 128), jnp.int32),
        pltpu.VMEM((40, 128), jnp.int32),
    ] + [pltpu.SemaphoreType.DMA] * (2 * NBUF),
)
def _agg_kernel(ya_hbm, yb_hbm, src_hbm, dst_hbm, oa_hbm, ob_hbm,
                acc, zv, rows, isv, idv, *sems):
    sg, ss = sems[:NBUF], sems[NBUF:]
    c = lax.axis_index("c")
    s = lax.axis_index("s")

    @pl.loop(0, 16)
    def _(i):
        @pl.loop(0, HD, step=16)
        def _(j):
            zv.at[pl.ds(i, 1), pl.ds(j, 16)][...] = jnp.zeros((1, 16), jnp.float32)

    @pl.loop(0, 40)
    def _(t):
        pltpu.sync_copy(zv, acc.at[pl.ds(s * 640 + t * 16, 16)])
    plsc.subcore_barrier()

    def run(tab, out):
        for p in range(2):
            pltpu.sync_copy(src_hbm.at[s].at[pl.ds(p * 40, 40)], isv)
            pltpu.sync_copy(dst_hbm.at[s].at[pl.ds(p * 40, 40)], idv)

            @pl.loop(0, 40, step=NBUF)
            def _(k0):
                gcs = [pltpu.async_copy(tab.at[isv.at[k0 + j]], rows.at[j],
                                        sg[j]) for j in range(NBUF)]
                scs = []
                for j in range(NBUF):
                    gcs[j].wait()
                    scs.append(pltpu.async_copy(rows.at[j],
                                                acc.at[idv.at[k0 + j]],
                                                ss[j], add=True))
                for cp in scs:
                    cp.wait()
        plsc.subcore_barrier()

        @pl.loop(0, 5)
        def _(t):
            r0 = s * 640 + t * 128
            pltpu.sync_copy(acc.at[pl.ds(r0, 128)], rows.at[0])
            pltpu.sync_copy(rows.at[0], out.at[pl.ds(r0, 128)])

    @pl.when(c == 0)
    def _():
        run(ya_hbm, oa_hbm)

    @pl.when(c == 1)
    def _():
        run(yb_hbm, ob_hbm)


def _lin1_body(x_ref, dp_ref, w_ref, ya_ref, yb_ref, dinv_ref):
    deg = dp_ref[0][:, 0:1] + dp_ref[1][:, 0:1] + 1.0
    dinv = lax.rsqrt(deg)
    xw = jnp.dot(x_ref[...], w_ref[...], preferred_element_type=jnp.float32)
    y = xw * dinv
    ya_ref[...] = y[:, :HD]
    yb_ref[...] = y[:, HD:]
    dinv_ref[...] = dinv


_lin1 = pl.pallas_call(
    _lin1_body,
    grid=(GRID,),
    in_specs=[
        pl.BlockSpec((BR, D), lambda i: (i, 0)),
        pl.BlockSpec((2, BR, 16), lambda i: (0, i, 0)),
        pl.BlockSpec((D, D), lambda i: (0, 0)),
    ],
    out_specs=[
        pl.BlockSpec((BR, HD), lambda i: (i, 0)),
        pl.BlockSpec((BR, HD), lambda i: (i, 0)),
        pl.BlockSpec((BR, 1), lambda i: (i, 0)),
    ],
    out_shape=[
        jax.ShapeDtypeStruct((NP, HD), jnp.float32),
        jax.ShapeDtypeStruct((NP, HD), jnp.float32),
        jax.ShapeDtypeStruct((NP, 1), jnp.float32),
    ],
)


def _lin2_body(aa_ref, ab_ref, ya_ref, yb_ref, dinv_ref, b1_ref, w2_ref,
               oa_ref, ob_ref):
    dinv = dinv_ref[...]
    g = jnp.concatenate([aa_ref[...] + ya_ref[...],
                         ab_ref[...] + yb_ref[...]], axis=1)
    h = jnp.maximum(g * dinv + b1_ref[...][None, :], 0.0)
    xw = jnp.dot(h, w2_ref[...], preferred_element_type=jnp.float32)
    y2 = xw * dinv
    oa_ref[...] = y2[:, :HD]
    ob_ref[...] = y2[:, HD:]


_lin2 = pl.pallas_call(
    _lin2_body,
    grid=(GRID,),
    in_specs=[
        pl.BlockSpec((BR, HD), lambda i: (i, 0)),
        pl.BlockSpec((BR, HD), lambda i: (i, 0)),
        pl.BlockSpec((BR, HD), lambda i: (i, 0)),
        pl.BlockSpec((BR, HD), lambda i: (i, 0)),
        pl.BlockSpec((BR, 1), lambda i: (i, 0)),
        pl.BlockSpec((D,), lambda i: (0,)),
        pl.BlockSpec((D, D), lambda i: (0, 0)),
    ],
    out_specs=[
        pl.BlockSpec((BR, HD), lambda i: (i, 0)),
        pl.BlockSpec((BR, HD), lambda i: (i, 0)),
    ],
    out_shape=[
        jax.ShapeDtypeStruct((NP, HD), jnp.float32),
        jax.ShapeDtypeStruct((NP, HD), jnp.float32),
    ],
)


def _pool_body(aa_ref, ab_ref, ya_ref, yb_ref, dinv_ref, b2_ref, batch_ref,
               out_ref):
    i = pl.program_id(0)
    dinv = dinv_ref[...]
    g = jnp.concatenate([aa_ref[...] + ya_ref[...],
                         ab_ref[...] + yb_ref[...]], axis=1)
    h = g * dinv + b2_ref[...][None, :]
    bt = batch_ref[...]
    m = (bt[None, :] == lax.broadcasted_iota(jnp.int32, (NG, BR), 0)
         ).astype(jnp.float32)

    @pl.when(i == 0)
    def _():
        out_ref[...] = jnp.zeros_like(out_ref)

    out_ref[...] += jnp.dot(m, h, preferred_element_type=jnp.float32)


_pool = pl.pallas_call(
    _pool_body,
    grid=(GRID,),
    in_specs=[
        pl.BlockSpec((BR, HD), lambda i: (i, 0)),
        pl.BlockSpec((BR, HD), lambda i: (i, 0)),
        pl.BlockSpec((BR, HD), lambda i: (i, 0)),
        pl.BlockSpec((BR, HD), lambda i: (i, 0)),
        pl.BlockSpec((BR, 1), lambda i: (i, 0)),
        pl.BlockSpec((D,), lambda i: (0,)),
        pl.BlockSpec((BR,), lambda i: (i,)),
    ],
    out_specs=pl.BlockSpec((NG, D), lambda i: (0, 0)),
    out_shape=jax.ShapeDtypeStruct((NG, D), jnp.float32),
)


def _head_body(p_ref, wfc_ref, bfc_ref, out_ref):
    z = jnp.dot(p_ref[...], wfc_ref[...], preferred_element_type=jnp.float32)
    z = z + bfc_ref[...][None, :]
    mx = jnp.max(z, axis=1, keepdims=True)
    lse = jnp.log(jnp.sum(jnp.exp(z - mx), axis=1, keepdims=True)) + mx
    out_ref[...] = z - lse


_head = pl.pallas_call(
    _head_body,
    grid=(1,),
    in_specs=[
        pl.BlockSpec((NG, D), lambda i: (0, 0)),
        pl.BlockSpec((D, NCLS), lambda i: (0, 0)),
        pl.BlockSpec((NCLS,), lambda i: (0,)),
    ],
    out_specs=pl.BlockSpec((NG, NCLS), lambda i: (0, 0)),
    out_shape=jax.ShapeDtypeStruct((NG, NCLS), jnp.float32),
)


def kernel(x, edge_index, batch, W1, b1, W2, b2, Wfc, bfc):
    src = edge_index[0]
    dst = edge_index[1]
    pad_e = EP - E
    src_p = jnp.concatenate([src, jnp.zeros((pad_e,), jnp.int32)])
    dst_p = jnp.concatenate(
        [dst, N + (jnp.arange(pad_e, dtype=jnp.int32) % (NP - N))])
    x_p = jnp.pad(x, ((0, NP - N), (0, 0)))
    batch_p = jnp.pad(batch, (0, NP - N), constant_values=NG)

    src3 = src_p.reshape(16, 80, 128)
    dst3 = dst_p.reshape(16, 80, 128)
    dst3d = dst_p.reshape(32, 40, 128)

    dp = _deg_kernel(dst3d)
    ya, yb, dinv = _lin1(x_p, dp, W1)
    a1a, a1b = _agg_kernel(ya, yb, src3, dst3)
    y2a, y2b = _lin2(a1a, a1b, ya, yb, dinv, b1, W2)
    a2a, a2b = _agg_kernel(y2a, y2b, src3, dst3)
    pooled = _pool(a2a, a2b, y2a, y2b, dinv, b2, batch_p)
    return _head(pooled, Wfc, bfc)


# speedup vs baseline: 8.3251x; 1.0388x over previous
import functools

import jax
import jax.numpy as jnp
from jax import lax
from jax.experimental import pallas as pl
from jax.experimental.pallas import tpu as pltpu
from jax.experimental.pallas import tpu_sc as plsc

N = 10000
E = 160000
D = 256
NCLS = 10
NG = 64
NP = 10240
EP = 163840
HD = 128
BR = 512
GRID = NP // BR

_mesh = plsc.VectorSubcoreMesh(core_axis_name="c", subcore_axis_name="s")


@functools.partial(
    pl.kernel,
    out_type=jax.ShapeDtypeStruct((2, NP, 16), jnp.float32),
    mesh=_mesh,
    scratch_types=[
        pltpu.VMEM_SHARED((NP, 16), jnp.float32),
        pltpu.VMEM((128, 16), jnp.float32),
        pltpu.VMEM((640, 16), jnp.float32),
        pltpu.VMEM((40, 128), jnp.int32),
        pltpu.SemaphoreType.DMA,
    ],
)
def _deg_kernel(dst_hbm, dp_hbm, acc, ones_v, wbuf, idx_v, sem):
    c = lax.axis_index("c")
    s = lax.axis_index("s")

    @pl.loop(0, 640)
    def _(i):
        wbuf.at[pl.ds(i, 1), :][...] = jnp.zeros((1, 16), jnp.float32)

    @pl.loop(0, 128)
    def _(i):
        ones_v.at[pl.ds(i, 1), :][...] = jnp.ones((1, 16), jnp.float32)

    pltpu.sync_copy(dst_hbm.at[c * 16 + s], idx_v)
    pltpu.sync_copy(wbuf, acc.at[pl.ds(s * 640, 640)])
    plsc.subcore_barrier()

    @pl.loop(0, 40, step=8)
    def _(k0):
        cps = [pltpu.async_copy(ones_v, acc.at[idx_v.at[k0 + j]], sem, add=True)
               for j in range(8)]
        for cp in cps:
            cp.wait()

    plsc.subcore_barrier()
    pltpu.sync_copy(acc.at[pl.ds(s * 640, 640)], wbuf)
    pltpu.sync_copy(wbuf, dp_hbm.at[c].at[pl.ds(s * 640, 640)])


NBUF = 4
CH = 64
NCH = 40
NPH = 4


@functools.partial(
    pl.kernel,
    out_type=[jax.ShapeDtypeStruct((NP, HD), jnp.float32),
              jax.ShapeDtypeStruct((NP, HD), jnp.float32)],
    mesh=_mesh,
    scratch_types=[
        pltpu.VMEM_SHARED((NP, HD), jnp.float32),
        pltpu.VMEM((NBUF, CH, HD), jnp.float32),
        pltpu.VMEM((NCH, CH), jnp.int32),
        pltpu.VMEM((NCH, CH), jnp.int32),
    ] + [pltpu.SemaphoreType.DMA] * (2 * NBUF),
)
def _agg_kernel(ya_hbm, yb_hbm, src_hbm, dst_hbm, oa_hbm, ob_hbm,
                acc, rows, isv, idv, *sems):
    sg, ss = sems[:NBUF], sems[NBUF:]
    c = lax.axis_index("c")
    s = lax.axis_index("s")

    @pl.loop(0, CH)
    def _(i):
        @pl.loop(0, HD, step=16)
        def _(j):
            rows.at[0, pl.ds(i, 1), pl.ds(j, 16)][...] = (
                jnp.zeros((1, 16), jnp.float32))

    @pl.loop(0, 10)
    def _(t):
        pltpu.sync_copy(rows.at[0], acc.at[pl.ds(s * 640 + t * CH, CH)])
    plsc.subcore_barrier()

    def run(tab, out):
        def wait_gather(j):
            pltpu.make_async_copy(tab.at[isv.at[0]], rows.at[j], sg[j]).wait()

        def wait_scatter(j):
            pltpu.make_async_copy(tab.at[isv.at[0]], rows.at[j], ss[j]).wait()

        for p in range(NPH):
            pltpu.sync_copy(src_hbm.at[s].at[pl.ds(p * NCH, NCH)], isv)
            pltpu.sync_copy(dst_hbm.at[s].at[pl.ds(p * NCH, NCH)], idv)
            for j in range(NBUF):
                pltpu.async_copy(tab.at[isv.at[j]], rows.at[j], sg[j])

            @pl.loop(0, NCH, step=NBUF)
            def _(k0):
                for j in range(NBUF):
                    wait_gather(j)
                    pltpu.async_copy(rows.at[j], acc.at[idv.at[k0 + j]],
                                     ss[j], add=True)
                for j in range(NBUF):
                    nk = k0 + NBUF + j

                    @pl.when(nk < NCH)
                    def _(nk=nk, j=j):
                        wait_scatter(j)
                        pltpu.async_copy(tab.at[isv.at[nk]], rows.at[j], sg[j])

            for j in range(NBUF):
                wait_scatter(j)
        plsc.subcore_barrier()

        @pl.loop(0, 10)
        def _(t):
            r0 = s * 640 + t * CH
            pltpu.sync_copy(acc.at[pl.ds(r0, CH)], rows.at[0])
            pltpu.sync_copy(rows.at[0], out.at[pl.ds(r0, CH)])

    @pl.when(c == 0)
    def _():
        run(ya_hbm, oa_hbm)

    @pl.when(c == 1)
    def _():
        run(yb_hbm, ob_hbm)


def _lin1_body(x_ref, dp_ref, w_ref, ya_ref, yb_ref, dinv_ref):
    deg = dp_ref[0][:, 0:1] + dp_ref[1][:, 0:1] + 1.0
    dinv = lax.rsqrt(deg)
    xw = jnp.dot(x_ref[...], w_ref[...], preferred_element_type=jnp.float32)
    y = xw * dinv
    ya_ref[...] = y[:, :HD]
    yb_ref[...] = y[:, HD:]
    dinv_ref[...] = dinv


_lin1 = pl.pallas_call(
    _lin1_body,
    grid=(GRID,),
    in_specs=[
        pl.BlockSpec((BR, D), lambda i: (i, 0)),
        pl.BlockSpec((2, BR, 16), lambda i: (0, i, 0)),
        pl.BlockSpec((D, D), lambda i: (0, 0)),
    ],
    out_specs=[
        pl.BlockSpec((BR, HD), lambda i: (i, 0)),
        pl.BlockSpec((BR, HD), lambda i: (i, 0)),
        pl.BlockSpec((BR, 1), lambda i: (i, 0)),
    ],
    out_shape=[
        jax.ShapeDtypeStruct((NP, HD), jnp.float32),
        jax.ShapeDtypeStruct((NP, HD), jnp.float32),
        jax.ShapeDtypeStruct((NP, 1), jnp.float32),
    ],
)


def _lin2_body(aa_ref, ab_ref, ya_ref, yb_ref, dinv_ref, b1_ref, w2_ref,
               oa_ref, ob_ref):
    dinv = dinv_ref[...]
    g = jnp.concatenate([aa_ref[...] + ya_ref[...],
                         ab_ref[...] + yb_ref[...]], axis=1)
    h = jnp.maximum(g * dinv + b1_ref[...][None, :], 0.0)
    xw = jnp.dot(h, w2_ref[...], preferred_element_type=jnp.float32)
    y2 = xw * dinv
    oa_ref[...] = y2[:, :HD]
    ob_ref[...] = y2[:, HD:]


_lin2 = pl.pallas_call(
    _lin2_body,
    grid=(GRID,),
    in_specs=[
        pl.BlockSpec((BR, HD), lambda i: (i, 0)),
        pl.BlockSpec((BR, HD), lambda i: (i, 0)),
        pl.BlockSpec((BR, HD), lambda i: (i, 0)),
        pl.BlockSpec((BR, HD), lambda i: (i, 0)),
        pl.BlockSpec((BR, 1), lambda i: (i, 0)),
        pl.BlockSpec((D,), lambda i: (0,)),
        pl.BlockSpec((D, D), lambda i: (0, 0)),
    ],
    out_specs=[
        pl.BlockSpec((BR, HD), lambda i: (i, 0)),
        pl.BlockSpec((BR, HD), lambda i: (i, 0)),
    ],
    out_shape=[
        jax.ShapeDtypeStruct((NP, HD), jnp.float32),
        jax.ShapeDtypeStruct((NP, HD), jnp.float32),
    ],
)


def _pool_body(aa_ref, ab_ref, ya_ref, yb_ref, dinv_ref, b2_ref, batch_ref,
               out_ref):
    i = pl.program_id(0)
    dinv = dinv_ref[...]
    g = jnp.concatenate([aa_ref[...] + ya_ref[...],
                         ab_ref[...] + yb_ref[...]], axis=1)
    h = g * dinv + b2_ref[...][None, :]
    bt = batch_ref[...]
    m = (bt[None, :] == lax.broadcasted_iota(jnp.int32, (NG, BR), 0)
         ).astype(jnp.float32)

    @pl.when(i == 0)
    def _():
        out_ref[...] = jnp.zeros_like(out_ref)

    out_ref[...] += jnp.dot(m, h, preferred_element_type=jnp.float32)


_pool = pl.pallas_call(
    _pool_body,
    grid=(GRID,),
    in_specs=[
        pl.BlockSpec((BR, HD), lambda i: (i, 0)),
        pl.BlockSpec((BR, HD), lambda i: (i, 0)),
        pl.BlockSpec((BR, HD), lambda i: (i, 0)),
        pl.BlockSpec((BR, HD), lambda i: (i, 0)),
        pl.BlockSpec((BR, 1), lambda i: (i, 0)),
        pl.BlockSpec((D,), lambda i: (0,)),
        pl.BlockSpec((BR,), lambda i: (i,)),
    ],
    out_specs=pl.BlockSpec((NG, D), lambda i: (0, 0)),
    out_shape=jax.ShapeDtypeStruct((NG, D), jnp.float32),
)


def _head_body(p_ref, wfc_ref, bfc_ref, out_ref):
    z = jnp.dot(p_ref[...], wfc_ref[...], preferred_element_type=jnp.float32)
    z = z + bfc_ref[...][None, :]
    mx = jnp.max(z, axis=1, keepdims=True)
    lse = jnp.log(jnp.sum(jnp.exp(z - mx), axis=1, keepdims=True)) + mx
    out_ref[...] = z - lse


_head = pl.pallas_call(
    _head_body,
    grid=(1,),
    in_specs=[
        pl.BlockSpec((NG, D), lambda i: (0, 0)),
        pl.BlockSpec((D, NCLS), lambda i: (0, 0)),
        pl.BlockSpec((NCLS,), lambda i: (0,)),
    ],
    out_specs=pl.BlockSpec((NG, NCLS), lambda i: (0, 0)),
    out_shape=jax.ShapeDtypeStruct((NG, NCLS), jnp.float32),
)


def kernel(x, edge_index, batch, W1, b1, W2, b2, Wfc, bfc):
    src = edge_index[0]
    dst = edge_index[1]
    pad_e = EP - E
    src_p = jnp.concatenate([src, jnp.zeros((pad_e,), jnp.int32)])
    dst_p = jnp.concatenate(
        [dst, N + (jnp.arange(pad_e, dtype=jnp.int32) % (NP - N))])
    x_p = jnp.pad(x, ((0, NP - N), (0, 0)))
    batch_p = jnp.pad(batch, (0, NP - N), constant_values=NG)

    src3 = src_p.reshape(16, NPH * NCH, CH)
    dst3 = dst_p.reshape(16, NPH * NCH, CH)
    dst3d = dst_p.reshape(32, 40, 128)

    dp = _deg_kernel(dst3d)
    ya, yb, dinv = _lin1(x_p, dp, W1)
    a1a, a1b = _agg_kernel(ya, yb, src3, dst3)
    y2a, y2b = _lin2(a1a, a1b, ya, yb, dinv, b1, W2)
    a2a, a2b = _agg_kernel(y2a, y2b, src3, dst3)
    pooled = _pool(a2a, a2b, y2a, y2b, dinv, b2, batch_p)
    return _head(pooled, Wfc, bfc)
